# Initial kernel scaffold; baseline (speedup 1.0000x reference)
import jax
import jax.numpy as jnp
from jax.experimental import pallas as pl


def kernel(features, edge_index, W1, b1, W2, b2):
    raise NotImplementedError("write your pallas kernel here")


# speedup vs baseline: 25.3744x; 25.3744x over previous
import jax
import jax.numpy as jnp
from jax import lax
from jax.experimental import pallas as pl
from jax.experimental.pallas import tpu as pltpu
from jax.experimental.pallas import tpu_sc as plsc

N_NODES = 10000
N_EDGES = 160000
IN_FEATS = 1433
HIDDEN = 16
OUT_FEATS = 7

NC = 2
NS = 16
NW = NC * NS
CH = 128
N_CHUNKS = 1280
E_PAD = N_CHUNKS * CH
CPW = N_CHUNKS // NW
N_ACC = 10240
ZERO_ROWS = N_ACC // NS

ROW_BLK = 1000


def _mm_body(x_ref, w_ref, o_ref):
    o_ref[...] = jnp.dot(x_ref[...], w_ref[...],
                         preferred_element_type=jnp.float32)


def _project1(features, W1):
    return pl.pallas_call(
        _mm_body,
        grid=(N_NODES // ROW_BLK,),
        in_specs=[
            pl.BlockSpec((ROW_BLK, IN_FEATS), lambda i: (i, 0)),
            pl.BlockSpec((IN_FEATS, HIDDEN), lambda i: (0, 0)),
        ],
        out_specs=pl.BlockSpec((ROW_BLK, HIDDEN), lambda i: (i, 0)),
        out_shape=jax.ShapeDtypeStruct((N_NODES, HIDDEN), jnp.float32),
    )(features, W1)


def _relu_project2_body(p_ref, b1_ref, w2_ref, o_ref):
    x = jnp.maximum(p_ref[0] + p_ref[1] + b1_ref[...], 0.0)
    o_ref[...] = jnp.dot(x, w2_ref[...], preferred_element_type=jnp.float32)


def _relu_project2(parts, b1, W2p):
    return pl.pallas_call(
        _relu_project2_body,
        grid=(N_NODES // ROW_BLK,),
        in_specs=[
            pl.BlockSpec((NC, ROW_BLK, HIDDEN), lambda i: (0, i, 0)),
            pl.BlockSpec((1, HIDDEN), lambda i: (0, 0)),
            pl.BlockSpec((HIDDEN, HIDDEN), lambda i: (0, 0)),
        ],
        out_specs=pl.BlockSpec((ROW_BLK, HIDDEN), lambda i: (i, 0)),
        out_shape=jax.ShapeDtypeStruct((N_NODES, HIDDEN), jnp.float32),
    )(parts, b1.reshape(1, HIDDEN), W2p)


def _combine_body(p_ref, b2_ref, o_ref):
    o_ref[...] = p_ref[0] + p_ref[1] + b2_ref[...]


def _combine(parts, b2p):
    return pl.pallas_call(
        _combine_body,
        grid=(N_NODES // ROW_BLK,),
        in_specs=[
            pl.BlockSpec((NC, ROW_BLK, HIDDEN), lambda i: (0, i, 0)),
            pl.BlockSpec((1, HIDDEN), lambda i: (0, 0)),
        ],
        out_specs=pl.BlockSpec((ROW_BLK, HIDDEN), lambda i: (i, 0)),
        out_shape=jax.ShapeDtypeStruct((N_NODES, HIDDEN), jnp.float32),
    )(parts, b2p.reshape(1, HIDDEN))


def _seg_sum_body(p_hbm, src_hbm, dst_hbm, zeros_hbm, out_hbm,
                  src_v, dst_v, rows_v, out_v, acc_sh, gsem):
    c = lax.axis_index("c")
    s = lax.axis_index("s")
    wid = s * NC + c

    pltpu.sync_copy(zeros_hbm, out_v)
    pltpu.sync_copy(out_v, acc_sh.at[pl.ds(s * ZERO_ROWS, ZERO_ROWS)])

    pltpu.sync_copy(src_hbm.at[pl.ds(wid * CPW, CPW)], src_v)
    pltpu.sync_copy(dst_hbm.at[pl.ds(wid * CPW, CPW)], dst_v)
    plsc.subcore_barrier()

    def fire(j, carry):
        pltpu.async_copy(p_hbm.at[src_v.at[j]], rows_v.at[j], gsem)
        return carry

    lax.fori_loop(0, CPW, fire, 0)

    def drain(j, carry):
        pltpu.make_async_copy(p_hbm.at[src_v.at[0]], rows_v.at[0], gsem).wait()
        return carry

    lax.fori_loop(0, CPW, drain, 0)

    def scat(j, carry):
        pltpu.sync_copy(rows_v.at[j], acc_sh.at[dst_v.at[j]], add=True)
        return carry

    lax.fori_loop(0, CPW, scat, 0)
    plsc.subcore_barrier()

    pltpu.sync_copy(acc_sh.at[pl.ds(s * ZERO_ROWS, ZERO_ROWS)], out_v)
    pltpu.sync_copy(out_v, out_hbm.at[c, pl.ds(s * ZERO_ROWS, ZERO_ROWS)])


_seg_sum_sc = pl.kernel(
    _seg_sum_body,
    out_type=jax.ShapeDtypeStruct((NC, N_ACC, HIDDEN), jnp.float32),
    mesh=plsc.VectorSubcoreMesh(core_axis_name="c", subcore_axis_name="s"),
    scratch_types=[
        pltpu.VMEM((CPW, CH), jnp.int32),
        pltpu.VMEM((CPW, CH), jnp.int32),
        pltpu.VMEM((CPW, CH, HIDDEN), jnp.float32),
        pltpu.VMEM((ZERO_ROWS, HIDDEN), jnp.float32),
        pltpu.VMEM_SHARED((N_ACC, HIDDEN), jnp.float32),
        pltpu.SemaphoreType.DMA,
    ],
    compiler_params=pltpu.CompilerParams(use_tc_tiling_on_sc=False),
)


def kernel(features, edge_index, W1, b1, W2, b2):
    src = edge_index[0].astype(jnp.int32)
    dst = edge_index[1].astype(jnp.int32)
    pad = E_PAD - N_EDGES
    src2d = jnp.concatenate(
        [src, jnp.zeros((pad,), jnp.int32)]).reshape(N_CHUNKS, CH)
    dst2d = jnp.concatenate(
        [dst, jnp.full((pad,), N_NODES, jnp.int32)]).reshape(N_CHUNKS, CH)
    zeros = jnp.zeros((ZERO_ROWS, HIDDEN), jnp.float32)
    W2p = jnp.pad(W2, ((0, 0), (0, HIDDEN - OUT_FEATS)))
    b2p = jnp.pad(b2, (0, HIDDEN - OUT_FEATS))

    p1 = _project1(features, W1)
    parts1 = _seg_sum_sc(p1, src2d, dst2d, zeros)
    p2 = _relu_project2(parts1, b1, W2p)
    parts2 = _seg_sum_sc(p2, src2d, dst2d, zeros)
    out16 = _combine(parts2, b2p)
    return out16[:, :OUT_FEATS]


# speedup vs baseline: 26.0206x; 1.0255x over previous
import jax
import jax.numpy as jnp
from jax import lax
from jax.experimental import pallas as pl
from jax.experimental.pallas import tpu as pltpu
from jax.experimental.pallas import tpu_sc as plsc

N_NODES = 10000
N_EDGES = 160000
IN_FEATS = 1433
HIDDEN = 16
OUT_FEATS = 7

NC = 2
NS = 16
NW = NC * NS
CH = 128
N_CHUNKS = 1280
E_PAD = N_CHUNKS * CH
CPW = N_CHUNKS // NW
N_ACC = 10240
ZERO_ROWS = N_ACC // NS

ROW_BLK = 1000


def _mm_body(x_ref, w_ref, o_ref):
    o_ref[...] = jnp.dot(x_ref[...], w_ref[...],
                         preferred_element_type=jnp.float32)


def _project1(features, W1):
    return pl.pallas_call(
        _mm_body,
        grid=(N_NODES // ROW_BLK,),
        in_specs=[
            pl.BlockSpec((ROW_BLK, IN_FEATS), lambda i: (i, 0)),
            pl.BlockSpec((IN_FEATS, HIDDEN), lambda i: (0, 0)),
        ],
        out_specs=pl.BlockSpec((ROW_BLK, HIDDEN), lambda i: (i, 0)),
        out_shape=jax.ShapeDtypeStruct((N_NODES, HIDDEN), jnp.float32),
    )(features, W1)


def _relu_project2_body(p_ref, b1_ref, w2_ref, o_ref):
    x = jnp.maximum(p_ref[0] + p_ref[1] + b1_ref[...], 0.0)
    o_ref[...] = jnp.dot(x, w2_ref[...], preferred_element_type=jnp.float32)


def _relu_project2(parts, b1, W2p):
    return pl.pallas_call(
        _relu_project2_body,
        grid=(N_NODES // ROW_BLK,),
        in_specs=[
            pl.BlockSpec((NC, ROW_BLK, HIDDEN), lambda i: (0, i, 0)),
            pl.BlockSpec((1, HIDDEN), lambda i: (0, 0)),
            pl.BlockSpec((HIDDEN, HIDDEN), lambda i: (0, 0)),
        ],
        out_specs=pl.BlockSpec((ROW_BLK, HIDDEN), lambda i: (i, 0)),
        out_shape=jax.ShapeDtypeStruct((N_NODES, HIDDEN), jnp.float32),
    )(parts, b1.reshape(1, HIDDEN), W2p)


def _combine_body(p_ref, b2_ref, o_ref):
    o_ref[...] = p_ref[0] + p_ref[1] + b2_ref[...]


def _combine(parts, b2p):
    return pl.pallas_call(
        _combine_body,
        grid=(N_NODES // ROW_BLK,),
        in_specs=[
            pl.BlockSpec((NC, ROW_BLK, HIDDEN), lambda i: (0, i, 0)),
            pl.BlockSpec((1, HIDDEN), lambda i: (0, 0)),
        ],
        out_specs=pl.BlockSpec((ROW_BLK, HIDDEN), lambda i: (i, 0)),
        out_shape=jax.ShapeDtypeStruct((N_NODES, HIDDEN), jnp.float32),
    )(parts, b2p.reshape(1, HIDDEN))


def _seg_sum_body(p_hbm, src_hbm, dst_hbm, zeros_hbm, out_hbm,
                  src_v, dst_v, rows_v, out_v, acc_sh, gsem, ssem):
    c = lax.axis_index("c")
    s = lax.axis_index("s")
    wid = s * NC + c

    pltpu.sync_copy(zeros_hbm, out_v)
    pltpu.sync_copy(out_v, acc_sh.at[pl.ds(s * ZERO_ROWS, ZERO_ROWS)])

    pltpu.sync_copy(src_hbm.at[pl.ds(wid * CPW, CPW)], src_v)
    pltpu.sync_copy(dst_hbm.at[pl.ds(wid * CPW, CPW)], dst_v)
    plsc.subcore_barrier()

    pltpu.async_copy(p_hbm.at[src_v.at[0]], rows_v.at[0], gsem)

    def step(i, carry):
        j0 = 2 * i
        pltpu.async_copy(p_hbm.at[src_v.at[j0 + 1]], rows_v.at[j0 + 1],
                         ssem)
        pltpu.make_async_copy(p_hbm.at[src_v.at[j0]], rows_v.at[j0],
                              gsem).wait()
        pltpu.sync_copy(rows_v.at[j0], acc_sh.at[dst_v.at[j0]], add=True)

        @pl.when(i < CPW // 2 - 1)
        def _():
            pltpu.async_copy(p_hbm.at[src_v.at[j0 + 2]],
                             rows_v.at[j0 + 2], gsem)

        pltpu.make_async_copy(p_hbm.at[src_v.at[j0 + 1]],
                              rows_v.at[j0 + 1], ssem).wait()
        pltpu.sync_copy(rows_v.at[j0 + 1], acc_sh.at[dst_v.at[j0 + 1]],
                        add=True)
        return carry

    lax.fori_loop(0, CPW // 2, step, 0)
    plsc.subcore_barrier()

    pltpu.sync_copy(acc_sh.at[pl.ds(s * ZERO_ROWS, ZERO_ROWS)], out_v)
    pltpu.sync_copy(out_v, out_hbm.at[c, pl.ds(s * ZERO_ROWS, ZERO_ROWS)])


_seg_sum_sc = pl.kernel(
    _seg_sum_body,
    out_type=jax.ShapeDtypeStruct((NC, N_ACC, HIDDEN), jnp.float32),
    mesh=plsc.VectorSubcoreMesh(core_axis_name="c", subcore_axis_name="s"),
    scratch_types=[
        pltpu.VMEM((CPW, CH), jnp.int32),
        pltpu.VMEM((CPW, CH), jnp.int32),
        pltpu.VMEM((CPW, CH, HIDDEN), jnp.float32),
        pltpu.VMEM((ZERO_ROWS, HIDDEN), jnp.float32),
        pltpu.VMEM_SHARED((N_ACC, HIDDEN), jnp.float32),
        pltpu.SemaphoreType.DMA,
        pltpu.SemaphoreType.DMA,
    ],
    compiler_params=pltpu.CompilerParams(use_tc_tiling_on_sc=False),
)


def kernel(features, edge_index, W1, b1, W2, b2):
    src = edge_index[0].astype(jnp.int32)
    dst = edge_index[1].astype(jnp.int32)
    pad = E_PAD - N_EDGES
    src2d = jnp.concatenate(
        [src, jnp.zeros((pad,), jnp.int32)]).reshape(N_CHUNKS, CH)
    dst2d = jnp.concatenate(
        [dst, jnp.full((pad,), N_NODES, jnp.int32)]).reshape(N_CHUNKS, CH)
    zeros = jnp.zeros((ZERO_ROWS, HIDDEN), jnp.float32)
    W2p = jnp.pad(W2, ((0, 0), (0, HIDDEN - OUT_FEATS)))
    b2p = jnp.pad(b2, (0, HIDDEN - OUT_FEATS))

    p1 = _project1(features, W1)
    parts1 = _seg_sum_sc(p1, src2d, dst2d, zeros)
    p2 = _relu_project2(parts1, b1, W2p)
    parts2 = _seg_sum_sc(p2, src2d, dst2d, zeros)
    out16 = _combine(parts2, b2p)
    return out16[:, :OUT_FEATS]


# speedup vs baseline: 32.6541x; 1.2549x over previous
import jax
import jax.numpy as jnp
from jax import lax
from jax.experimental import pallas as pl
from jax.experimental.pallas import tpu as pltpu
from jax.experimental.pallas import tpu_sc as plsc

N_NODES = 10000
N_EDGES = 160000
IN_FEATS = 1433
HIDDEN = 16
OUT_FEATS = 7

NC = 2
NS = 16
NW = NC * NS
CH = 128
N_CHUNKS = 1280
E_PAD = N_CHUNKS * CH
CPW = N_CHUNKS // NW
N_ACC = 10240
ZERO_ROWS = N_ACC // NS
P_ROWS = N_NODES // NS

ROW_BLK = 1000


def _mm_body(x_ref, w_ref, o_ref):
    o_ref[...] = jnp.dot(x_ref[...], w_ref[...],
                         preferred_element_type=jnp.float32)


def _project1(features, W1):
    return pl.pallas_call(
        _mm_body,
        grid=(N_NODES // ROW_BLK,),
        in_specs=[
            pl.BlockSpec((ROW_BLK, IN_FEATS), lambda i: (i, 0)),
            pl.BlockSpec((IN_FEATS, HIDDEN), lambda i: (0, 0)),
        ],
        out_specs=pl.BlockSpec((ROW_BLK, HIDDEN), lambda i: (i, 0)),
        out_shape=jax.ShapeDtypeStruct((N_NODES, HIDDEN), jnp.float32),
    )(features, W1)


def _relu_project2_body(p_ref, b1_ref, w2_ref, o_ref):
    x = jnp.maximum(p_ref[0] + p_ref[1] + b1_ref[...], 0.0)
    o_ref[...] = jnp.dot(x, w2_ref[...], preferred_element_type=jnp.float32)


def _relu_project2(parts, b1, W2p):
    return pl.pallas_call(
        _relu_project2_body,
        grid=(N_NODES // ROW_BLK,),
        in_specs=[
            pl.BlockSpec((NC, ROW_BLK, HIDDEN), lambda i: (0, i, 0)),
            pl.BlockSpec((1, HIDDEN), lambda i: (0, 0)),
            pl.BlockSpec((HIDDEN, HIDDEN), lambda i: (0, 0)),
        ],
        out_specs=pl.BlockSpec((ROW_BLK, HIDDEN), lambda i: (i, 0)),
        out_shape=jax.ShapeDtypeStruct((N_NODES, HIDDEN), jnp.float32),
    )(parts, b1.reshape(1, HIDDEN), W2p)


def _combine_body(p_ref, b2_ref, o_ref):
    o_ref[...] = p_ref[0] + p_ref[1] + b2_ref[...]


def _combine(parts, b2p):
    return pl.pallas_call(
        _combine_body,
        grid=(N_NODES // ROW_BLK,),
        in_specs=[
            pl.BlockSpec((NC, ROW_BLK, HIDDEN), lambda i: (0, i, 0)),
            pl.BlockSpec((1, HIDDEN), lambda i: (0, 0)),
        ],
        out_specs=pl.BlockSpec((ROW_BLK, HIDDEN), lambda i: (i, 0)),
        out_shape=jax.ShapeDtypeStruct((N_NODES, HIDDEN), jnp.float32),
    )(parts, b2p.reshape(1, HIDDEN))


def _seg_sum_body(p_hbm, src_hbm, dst_hbm, zeros_hbm, out_hbm,
                  src_v, dst_v, rows_v, out_v, acc_sh, tab_sh, gsem, ssem):
    c = lax.axis_index("c")
    s = lax.axis_index("s")
    wid = s * NC + c

    pltpu.sync_copy(zeros_hbm, out_v)
    pltpu.sync_copy(out_v, acc_sh.at[pl.ds(s * ZERO_ROWS, ZERO_ROWS)])
    pltpu.sync_copy(p_hbm.at[pl.ds(s * P_ROWS, P_ROWS)],
                    tab_sh.at[pl.ds(s * P_ROWS, P_ROWS)])

    pltpu.sync_copy(src_hbm.at[pl.ds(wid * CPW, CPW)], src_v)
    pltpu.sync_copy(dst_hbm.at[pl.ds(wid * CPW, CPW)], dst_v)
    plsc.subcore_barrier()

    pltpu.async_copy(tab_sh.at[src_v.at[0]], rows_v.at[0], gsem)

    def step(i, carry):
        j0 = 2 * i
        pltpu.async_copy(tab_sh.at[src_v.at[j0 + 1]], rows_v.at[j0 + 1],
                         ssem)
        pltpu.make_async_copy(tab_sh.at[src_v.at[j0]], rows_v.at[j0],
                              gsem).wait()
        pltpu.sync_copy(rows_v.at[j0], acc_sh.at[dst_v.at[j0]], add=True)

        @pl.when(i < CPW // 2 - 1)
        def _():
            pltpu.async_copy(tab_sh.at[src_v.at[j0 + 2]],
                             rows_v.at[j0 + 2], gsem)

        pltpu.make_async_copy(tab_sh.at[src_v.at[j0 + 1]],
                              rows_v.at[j0 + 1], ssem).wait()
        pltpu.sync_copy(rows_v.at[j0 + 1], acc_sh.at[dst_v.at[j0 + 1]],
                        add=True)
        return carry

    lax.fori_loop(0, CPW // 2, step, 0)
    plsc.subcore_barrier()

    pltpu.sync_copy(acc_sh.at[pl.ds(s * ZERO_ROWS, ZERO_ROWS)], out_v)
    pltpu.sync_copy(out_v, out_hbm.at[c, pl.ds(s * ZERO_ROWS, ZERO_ROWS)])


_seg_sum_sc = pl.kernel(
    _seg_sum_body,
    out_type=jax.ShapeDtypeStruct((NC, N_ACC, HIDDEN), jnp.float32),
    mesh=plsc.VectorSubcoreMesh(core_axis_name="c", subcore_axis_name="s"),
    scratch_types=[
        pltpu.VMEM((CPW, CH), jnp.int32),
        pltpu.VMEM((CPW, CH), jnp.int32),
        pltpu.VMEM((CPW, CH, HIDDEN), jnp.float32),
        pltpu.VMEM((ZERO_ROWS, HIDDEN), jnp.float32),
        pltpu.VMEM_SHARED((N_ACC, HIDDEN), jnp.float32),
        pltpu.VMEM_SHARED((N_NODES, HIDDEN), jnp.float32),
        pltpu.SemaphoreType.DMA,
        pltpu.SemaphoreType.DMA,
    ],
    compiler_params=pltpu.CompilerParams(use_tc_tiling_on_sc=False),
)


def kernel(features, edge_index, W1, b1, W2, b2):
    src = edge_index[0].astype(jnp.int32)
    dst = edge_index[1].astype(jnp.int32)
    pad = E_PAD - N_EDGES
    src2d = jnp.concatenate(
        [src, jnp.zeros((pad,), jnp.int32)]).reshape(N_CHUNKS, CH)
    dst2d = jnp.concatenate(
        [dst, jnp.full((pad,), N_NODES, jnp.int32)]).reshape(N_CHUNKS, CH)
    zeros = jnp.zeros((ZERO_ROWS, HIDDEN), jnp.float32)
    W2p = jnp.pad(W2, ((0, 0), (0, HIDDEN - OUT_FEATS)))
    b2p = jnp.pad(b2, (0, HIDDEN - OUT_FEATS))

    p1 = _project1(features, W1)
    parts1 = _seg_sum_sc(p1, src2d, dst2d, zeros)
    p2 = _relu_project2(parts1, b1, W2p)
    parts2 = _seg_sum_sc(p2, src2d, dst2d, zeros)
    out16 = _combine(parts2, b2p)
    return out16[:, :OUT_FEATS]


# speedup vs baseline: 35.0333x; 1.0729x over previous
import jax
import jax.numpy as jnp
from jax import lax
from jax.experimental import pallas as pl
from jax.experimental.pallas import tpu as pltpu
from jax.experimental.pallas import tpu_sc as plsc

N_NODES = 10000
N_EDGES = 160000
IN_FEATS = 1433
HIDDEN = 16
OUT_FEATS = 7

NC = 2
NS = 16
NW = NC * NS
CH = 128
N_CHUNKS = 1280
E_PAD = N_CHUNKS * CH
CPW = N_CHUNKS // NW
NBUF = 4
N_ACC = 10240
ZERO_ROWS = N_ACC // NS
P_ROWS = N_NODES // NS

ROW_BLK = 1000

_SC_MESH = plsc.VectorSubcoreMesh(core_axis_name="c", subcore_axis_name="s")


def _mm_body(x_ref, w_ref, o_ref):
    o_ref[...] = jnp.dot(x_ref[...], w_ref[...],
                         preferred_element_type=jnp.float32)


def _project1(features, W1):
    return pl.pallas_call(
        _mm_body,
        grid=(N_NODES // ROW_BLK,),
        in_specs=[
            pl.BlockSpec((ROW_BLK, IN_FEATS), lambda i: (i, 0)),
            pl.BlockSpec((IN_FEATS, HIDDEN), lambda i: (0, 0)),
        ],
        out_specs=pl.BlockSpec((ROW_BLK, HIDDEN), lambda i: (i, 0)),
        out_shape=jax.ShapeDtypeStruct((N_NODES, HIDDEN), jnp.float32),
    )(features, W1)


def _final_body(p_ref, w2_ref, b2_ref, o_ref):
    o_ref[...] = jnp.dot(p_ref[0] + p_ref[1], w2_ref[...],
                         preferred_element_type=jnp.float32,
                         precision=lax.Precision.HIGHEST) + b2_ref[...]


def _final(parts, W2p, b2p):
    return pl.pallas_call(
        _final_body,
        grid=(N_NODES // ROW_BLK,),
        in_specs=[
            pl.BlockSpec((NC, ROW_BLK, HIDDEN), lambda i: (0, i, 0)),
            pl.BlockSpec((HIDDEN, HIDDEN), lambda i: (0, 0)),
            pl.BlockSpec((1, HIDDEN), lambda i: (0, 0)),
        ],
        out_specs=pl.BlockSpec((ROW_BLK, HIDDEN), lambda i: (i, 0)),
        out_shape=jax.ShapeDtypeStruct((N_NODES, HIDDEN), jnp.float32),
    )(parts, W2p, b2p.reshape(1, HIDDEN))


def _edge_phase(s, wid, src_hbm, dst_hbm, out_hbm,
                src_v, dst_v, rows_v, out_v, acc_sh, tab_sh, gsem, ssem):
    pltpu.sync_copy(src_hbm.at[pl.ds(wid * CPW, CPW)], src_v)
    pltpu.sync_copy(dst_hbm.at[pl.ds(wid * CPW, CPW)], dst_v)
    plsc.subcore_barrier()

    pltpu.async_copy(tab_sh.at[src_v.at[0]], rows_v.at[0], gsem)

    def step(i, carry):
        j0 = 2 * i
        b0 = lax.rem(j0, NBUF)
        b1_ = lax.rem(j0 + 1, NBUF)
        b2_ = lax.rem(j0 + 2, NBUF)
        pltpu.async_copy(tab_sh.at[src_v.at[j0 + 1]], rows_v.at[b1_],
                         ssem)
        pltpu.make_async_copy(tab_sh.at[src_v.at[j0]], rows_v.at[b0],
                              gsem).wait()
        pltpu.sync_copy(rows_v.at[b0], acc_sh.at[dst_v.at[j0]], add=True)

        @pl.when(i < CPW // 2 - 1)
        def _():
            pltpu.async_copy(tab_sh.at[src_v.at[j0 + 2]],
                             rows_v.at[b2_], gsem)

        pltpu.make_async_copy(tab_sh.at[src_v.at[j0 + 1]],
                              rows_v.at[b1_], ssem).wait()
        pltpu.sync_copy(rows_v.at[b1_], acc_sh.at[dst_v.at[j0 + 1]],
                        add=True)
        return carry

    lax.fori_loop(0, CPW // 2, step, 0)
    plsc.subcore_barrier()

    pltpu.sync_copy(acc_sh.at[pl.ds(s * ZERO_ROWS, ZERO_ROWS)], out_v)
    c = wid % NC
    pltpu.sync_copy(out_v, out_hbm.at[c, pl.ds(s * ZERO_ROWS, ZERO_ROWS)])


def _seg_sum1_body(p_hbm, src_hbm, dst_hbm, zeros_hbm, out_hbm,
                   src_v, dst_v, rows_v, out_v, acc_sh, tab_sh, gsem, ssem):
    c = lax.axis_index("c")
    s = lax.axis_index("s")
    pltpu.sync_copy(zeros_hbm, out_v)
    pltpu.sync_copy(out_v, acc_sh.at[pl.ds(s * ZERO_ROWS, ZERO_ROWS)])
    pltpu.sync_copy(p_hbm.at[pl.ds(s * P_ROWS, P_ROWS)],
                    tab_sh.at[pl.ds(s * P_ROWS, P_ROWS)])
    _edge_phase(s, s * NC + c, src_hbm, dst_hbm, out_hbm,
                src_v, dst_v, rows_v, out_v, acc_sh, tab_sh, gsem, ssem)


def _seg_sum2_body(parts_hbm, b1_hbm, src_hbm, dst_hbm, zeros_hbm, out_hbm,
                   src_v, dst_v, rows_v, out_v, a_v, b_v, b1_v,
                   acc_sh, tab_sh, gsem, ssem):
    c = lax.axis_index("c")
    s = lax.axis_index("s")
    pltpu.sync_copy(zeros_hbm, out_v)
    pltpu.sync_copy(out_v, acc_sh.at[pl.ds(s * ZERO_ROWS, ZERO_ROWS)])

    pltpu.sync_copy(parts_hbm.at[0, pl.ds(s * P_ROWS, P_ROWS)], a_v)
    pltpu.sync_copy(parts_hbm.at[1, pl.ds(s * P_ROWS, P_ROWS)], b_v)
    pltpu.sync_copy(b1_hbm, b1_v)
    b1vec = b1_v[...]

    def rowf(r, carry):
        out_v[r, :] = jnp.maximum(a_v[r, :] + b_v[r, :] + b1vec, 0.0)
        return carry

    lax.fori_loop(0, P_ROWS, rowf, 0)
    pltpu.sync_copy(out_v.at[pl.ds(0, P_ROWS)],
                    tab_sh.at[pl.ds(s * P_ROWS, P_ROWS)])
    _edge_phase(s, s * NC + c, src_hbm, dst_hbm, out_hbm,
                src_v, dst_v, rows_v, out_v, acc_sh, tab_sh, gsem, ssem)


_COMMON_SCRATCH = [
    pltpu.VMEM((CPW, CH), jnp.int32),
    pltpu.VMEM((CPW, CH), jnp.int32),
    pltpu.VMEM((NBUF, CH, HIDDEN), jnp.float32),
    pltpu.VMEM((ZERO_ROWS, HIDDEN), jnp.float32),
]
_SHARED_SCRATCH = [
    pltpu.VMEM_SHARED((N_ACC, HIDDEN), jnp.float32),
    pltpu.VMEM_SHARED((N_NODES, HIDDEN), jnp.float32),
    pltpu.SemaphoreType.DMA,
    pltpu.SemaphoreType.DMA,
]

_seg_sum1 = pl.kernel(
    _seg_sum1_body,
    out_type=jax.ShapeDtypeStruct((NC, N_ACC, HIDDEN), jnp.float32),
    mesh=_SC_MESH,
    scratch_types=_COMMON_SCRATCH + _SHARED_SCRATCH,
    compiler_params=pltpu.CompilerParams(use_tc_tiling_on_sc=False),
)

_seg_sum2 = pl.kernel(
    _seg_sum2_body,
    out_type=jax.ShapeDtypeStruct((NC, N_ACC, HIDDEN), jnp.float32),
    mesh=_SC_MESH,
    scratch_types=_COMMON_SCRATCH + [
        pltpu.VMEM((P_ROWS, HIDDEN), jnp.float32),
        pltpu.VMEM((P_ROWS, HIDDEN), jnp.float32),
        pltpu.VMEM((HIDDEN,), jnp.float32),
    ] + _SHARED_SCRATCH,
    compiler_params=pltpu.CompilerParams(use_tc_tiling_on_sc=False),
)


def kernel(features, edge_index, W1, b1, W2, b2):
    src = edge_index[0].astype(jnp.int32)
    dst = edge_index[1].astype(jnp.int32)
    pad = E_PAD - N_EDGES
    src2d = jnp.concatenate(
        [src, jnp.zeros((pad,), jnp.int32)]).reshape(N_CHUNKS, CH)
    dst2d = jnp.concatenate(
        [dst, jnp.full((pad,), N_NODES, jnp.int32)]).reshape(N_CHUNKS, CH)
    zeros = jnp.zeros((ZERO_ROWS, HIDDEN), jnp.float32)
    W2p = jnp.pad(W2, ((0, 0), (0, HIDDEN - OUT_FEATS)))
    b2p = jnp.pad(b2, (0, HIDDEN - OUT_FEATS))

    p1 = _project1(features, W1)
    parts1 = _seg_sum1(p1, src2d, dst2d, zeros)
    parts2 = _seg_sum2(parts1, b1, src2d, dst2d, zeros)
    out16 = _final(parts2, W2p, b2p)
    return out16[:, :OUT_FEATS]


# speedup vs baseline: 36.9435x; 1.0545x over previous
import jax
import jax.numpy as jnp
from jax import lax
from jax.experimental import pallas as pl
from jax.experimental.pallas import tpu as pltpu
from jax.experimental.pallas import tpu_sc as plsc

N_NODES = 10000
N_EDGES = 160000
IN_FEATS = 1433
HIDDEN = 16
OUT_FEATS = 7

NC = 2
NS = 16
NW = NC * NS
CH = 128
N_CHUNKS = 1280
E_PAD = N_CHUNKS * CH
CPW = N_CHUNKS // NW
NBUF = 4
N_ACC = 10240
ZERO_ROWS = N_ACC // NS
P_ROWS = N_NODES // NS

ROW_BLK = 1000

_SC_MESH = plsc.VectorSubcoreMesh(core_axis_name="c", subcore_axis_name="s")


def _mm_body(x_ref, w_ref, o_ref):
    o_ref[...] = jnp.dot(x_ref[...], w_ref[...],
                         preferred_element_type=jnp.float32)


def _project1(features, W1):
    return pl.pallas_call(
        _mm_body,
        grid=(N_NODES // ROW_BLK,),
        in_specs=[
            pl.BlockSpec((ROW_BLK, IN_FEATS), lambda i: (i, 0)),
            pl.BlockSpec((IN_FEATS, HIDDEN), lambda i: (0, 0)),
        ],
        out_specs=pl.BlockSpec((ROW_BLK, HIDDEN), lambda i: (i, 0)),
        out_shape=jax.ShapeDtypeStruct((N_NODES, HIDDEN), jnp.float32),
    )(features, W1)


def _final_body(p_ref, w2_ref, b2_ref, o_ref):
    o_ref[...] = jnp.dot(p_ref[0] + p_ref[1], w2_ref[...],
                         preferred_element_type=jnp.float32,
                         precision=lax.Precision.HIGHEST) + b2_ref[...]


PK_ROWS = N_ACC // 8
PK_BLK = 128


def _final(parts_pk, W2blk, b2t):
    return pl.pallas_call(
        _final_body,
        grid=(PK_ROWS // PK_BLK,),
        in_specs=[
            pl.BlockSpec((NC, PK_BLK, 128), lambda i: (0, i, 0)),
            pl.BlockSpec((128, 128), lambda i: (0, 0)),
            pl.BlockSpec((1, 128), lambda i: (0, 0)),
        ],
        out_specs=pl.BlockSpec((PK_BLK, 128), lambda i: (i, 0)),
        out_shape=jax.ShapeDtypeStruct((PK_ROWS, 128), jnp.float32),
    )(parts_pk, W2blk, b2t.reshape(1, 128))


def _edge_phase(s, wid, src_hbm, dst_hbm, out_hbm,
                src_v, dst_v, rows_v, out_v, acc_sh, tab_sh, gsem, ssem):
    pltpu.sync_copy(src_hbm.at[pl.ds(wid * CPW, CPW)], src_v)
    pltpu.sync_copy(dst_hbm.at[pl.ds(wid * CPW, CPW)], dst_v)
    plsc.subcore_barrier()

    pltpu.async_copy(tab_sh.at[src_v.at[0]], rows_v.at[0], gsem)

    def step(i, carry):
        j0 = 2 * i
        b0 = lax.rem(j0, NBUF)
        b1_ = lax.rem(j0 + 1, NBUF)
        b2_ = lax.rem(j0 + 2, NBUF)
        pltpu.async_copy(tab_sh.at[src_v.at[j0 + 1]], rows_v.at[b1_],
                         ssem)
        pltpu.make_async_copy(tab_sh.at[src_v.at[j0]], rows_v.at[b0],
                              gsem).wait()
        pltpu.sync_copy(rows_v.at[b0], acc_sh.at[dst_v.at[j0]], add=True)

        @pl.when(i < CPW // 2 - 1)
        def _():
            pltpu.async_copy(tab_sh.at[src_v.at[j0 + 2]],
                             rows_v.at[b2_], gsem)

        pltpu.make_async_copy(tab_sh.at[src_v.at[j0 + 1]],
                              rows_v.at[b1_], ssem).wait()
        pltpu.sync_copy(rows_v.at[b1_], acc_sh.at[dst_v.at[j0 + 1]],
                        add=True)
        return carry

    lax.fori_loop(0, CPW // 2, step, 0)
    plsc.subcore_barrier()

    pltpu.sync_copy(acc_sh.at[pl.ds(s * ZERO_ROWS, ZERO_ROWS)], out_v)
    c = wid % NC
    pltpu.sync_copy(out_v, out_hbm.at[c, pl.ds(s * ZERO_ROWS, ZERO_ROWS)])


def _seg_sum1_body(p_hbm, src_hbm, dst_hbm, zeros_hbm, out_hbm,
                   src_v, dst_v, rows_v, out_v, acc_sh, tab_sh, gsem, ssem):
    c = lax.axis_index("c")
    s = lax.axis_index("s")
    pltpu.sync_copy(zeros_hbm, out_v)
    pltpu.sync_copy(out_v, acc_sh.at[pl.ds(s * ZERO_ROWS, ZERO_ROWS)])
    pltpu.sync_copy(p_hbm.at[pl.ds(s * P_ROWS, P_ROWS)],
                    tab_sh.at[pl.ds(s * P_ROWS, P_ROWS)])
    _edge_phase(s, s * NC + c, src_hbm, dst_hbm, out_hbm,
                src_v, dst_v, rows_v, out_v, acc_sh, tab_sh, gsem, ssem)


def _seg_sum2_body(parts_hbm, b1_hbm, src_hbm, dst_hbm, zeros_hbm, out_hbm,
                   src_v, dst_v, rows_v, out_v, a_v, b_v, b1_v,
                   acc_sh, tab_sh, gsem, ssem):
    c = lax.axis_index("c")
    s = lax.axis_index("s")
    pltpu.sync_copy(zeros_hbm, out_v)
    pltpu.sync_copy(out_v, acc_sh.at[pl.ds(s * ZERO_ROWS, ZERO_ROWS)])

    pltpu.sync_copy(parts_hbm.at[0, pl.ds(s * P_ROWS, P_ROWS)], a_v)
    pltpu.sync_copy(parts_hbm.at[1, pl.ds(s * P_ROWS, P_ROWS)], b_v)
    pltpu.sync_copy(b1_hbm, b1_v)
    b1vec = b1_v[...]

    def rowf(r, carry):
        out_v[r, :] = jnp.maximum(a_v[r, :] + b_v[r, :] + b1vec, 0.0)
        return carry

    lax.fori_loop(0, P_ROWS, rowf, 0)
    pltpu.sync_copy(out_v.at[pl.ds(0, P_ROWS)],
                    tab_sh.at[pl.ds(s * P_ROWS, P_ROWS)])
    _edge_phase(s, s * NC + c, src_hbm, dst_hbm, out_hbm,
                src_v, dst_v, rows_v, out_v, acc_sh, tab_sh, gsem, ssem)


_COMMON_SCRATCH = [
    pltpu.VMEM((CPW, CH), jnp.int32),
    pltpu.VMEM((CPW, CH), jnp.int32),
    pltpu.VMEM((NBUF, CH, HIDDEN), jnp.float32),
    pltpu.VMEM((ZERO_ROWS, HIDDEN), jnp.float32),
]
_SHARED_SCRATCH = [
    pltpu.VMEM_SHARED((N_ACC, HIDDEN), jnp.float32),
    pltpu.VMEM_SHARED((N_NODES, HIDDEN), jnp.float32),
    pltpu.SemaphoreType.DMA,
    pltpu.SemaphoreType.DMA,
]

_seg_sum1 = pl.kernel(
    _seg_sum1_body,
    out_type=jax.ShapeDtypeStruct((NC, N_ACC, HIDDEN), jnp.float32),
    mesh=_SC_MESH,
    scratch_types=_COMMON_SCRATCH + _SHARED_SCRATCH,
    compiler_params=pltpu.CompilerParams(use_tc_tiling_on_sc=False),
)

_seg_sum2 = pl.kernel(
    _seg_sum2_body,
    out_type=jax.ShapeDtypeStruct((NC, N_ACC, HIDDEN), jnp.float32),
    mesh=_SC_MESH,
    scratch_types=_COMMON_SCRATCH + [
        pltpu.VMEM((P_ROWS, HIDDEN), jnp.float32),
        pltpu.VMEM((P_ROWS, HIDDEN), jnp.float32),
        pltpu.VMEM((HIDDEN,), jnp.float32),
    ] + _SHARED_SCRATCH,
    compiler_params=pltpu.CompilerParams(use_tc_tiling_on_sc=False),
)


def kernel(features, edge_index, W1, b1, W2, b2):
    src = edge_index[0].astype(jnp.int32)
    dst = edge_index[1].astype(jnp.int32)
    pad = E_PAD - N_EDGES
    src2d = jnp.concatenate(
        [src, jnp.zeros((pad,), jnp.int32)]).reshape(N_CHUNKS, CH)
    dst2d = jnp.concatenate(
        [dst, jnp.full((pad,), N_NODES, jnp.int32)]).reshape(N_CHUNKS, CH)
    zeros = jnp.zeros((ZERO_ROWS, HIDDEN), jnp.float32)
    W2p = jnp.pad(W2, ((0, 0), (0, HIDDEN - OUT_FEATS)))
    b2p = jnp.pad(b2, (0, HIDDEN - OUT_FEATS))

    W2blk = jnp.kron(jnp.eye(8, dtype=jnp.float32), W2p)
    b2t = jnp.tile(b2p, 8)

    p1 = _project1(features, W1)
    parts1 = _seg_sum1(p1, src2d, dst2d, zeros)
    parts2 = _seg_sum2(parts1, b1, src2d, dst2d, zeros)
    parts_pk = parts2.reshape(NC, PK_ROWS, 128)
    out_pk = _final(parts_pk, W2blk, b2t)
    return out_pk.reshape(N_ACC, HIDDEN)[:N_NODES, :OUT_FEATS]


# speedup vs baseline: 37.4456x; 1.0136x over previous
import jax
import jax.numpy as jnp
from jax import lax
from jax.experimental import pallas as pl
from jax.experimental.pallas import tpu as pltpu
from jax.experimental.pallas import tpu_sc as plsc

N_NODES = 10000
N_EDGES = 160000
IN_FEATS = 1433
HIDDEN = 16
OUT_FEATS = 7

NC = 2
NS = 16
NW = NC * NS
CH = 128
N_CHUNKS = 1280
E_PAD = N_CHUNKS * CH
CPW = N_CHUNKS // NW
NBUF = 4
N_ACC = 10240
ZERO_ROWS = N_ACC // NS
P_ROWS = N_NODES // NS

ROW_BLK = 1000

_SC_MESH = plsc.VectorSubcoreMesh(core_axis_name="c", subcore_axis_name="s")


def _mm_body(x_ref, w_ref, o_ref):
    o_ref[...] = jnp.dot(x_ref[...], w_ref[...],
                         preferred_element_type=jnp.float32)


def _project1(features, W1p):
    return pl.pallas_call(
        _mm_body,
        grid=(N_NODES // ROW_BLK,),
        in_specs=[
            pl.BlockSpec((ROW_BLK, IN_FEATS), lambda i: (i, 0)),
            pl.BlockSpec((IN_FEATS, 128), lambda i: (0, 0)),
        ],
        out_specs=pl.BlockSpec((ROW_BLK, 128), lambda i: (i, 0)),
        out_shape=jax.ShapeDtypeStruct((N_NODES, 128), jnp.float32),
    )(features, W1p)


def _final_body(p_ref, w2_ref, b2_ref, o_ref):
    o_ref[...] = jnp.dot(p_ref[0] + p_ref[1], w2_ref[...],
                         preferred_element_type=jnp.float32,
                         precision=lax.Precision.HIGHEST) + b2_ref[...]


PK_ROWS = N_ACC // 8
PK_BLK = 128


def _final(parts_pk, W2blk, b2t):
    return pl.pallas_call(
        _final_body,
        grid=(PK_ROWS // PK_BLK,),
        in_specs=[
            pl.BlockSpec((NC, PK_BLK, 128), lambda i: (0, i, 0)),
            pl.BlockSpec((128, 128), lambda i: (0, 0)),
            pl.BlockSpec((1, 128), lambda i: (0, 0)),
        ],
        out_specs=pl.BlockSpec((PK_BLK, 128), lambda i: (i, 0)),
        out_shape=jax.ShapeDtypeStruct((PK_ROWS, 128), jnp.float32),
    )(parts_pk, W2blk, b2t.reshape(1, 128))


def _edge_phase(s, wid, src_hbm, dst_hbm, out_hbm,
                src_v, dst_v, rows_v, out_v, acc_sh, tab_sh, gsem, ssem):
    pltpu.sync_copy(src_hbm.at[pl.ds(wid * CPW, CPW)], src_v)
    pltpu.sync_copy(dst_hbm.at[pl.ds(wid * CPW, CPW)], dst_v)
    plsc.subcore_barrier()

    pltpu.async_copy(tab_sh.at[src_v.at[0]], rows_v.at[0], gsem)

    def step(i, carry):
        j0 = 2 * i
        b0 = lax.rem(j0, NBUF)
        b1_ = lax.rem(j0 + 1, NBUF)
        b2_ = lax.rem(j0 + 2, NBUF)
        pltpu.async_copy(tab_sh.at[src_v.at[j0 + 1]], rows_v.at[b1_],
                         ssem)
        pltpu.make_async_copy(tab_sh.at[src_v.at[j0]], rows_v.at[b0],
                              gsem).wait()
        pltpu.sync_copy(rows_v.at[b0], acc_sh.at[dst_v.at[j0]], add=True)

        @pl.when(i < CPW // 2 - 1)
        def _():
            pltpu.async_copy(tab_sh.at[src_v.at[j0 + 2]],
                             rows_v.at[b2_], gsem)

        pltpu.make_async_copy(tab_sh.at[src_v.at[j0 + 1]],
                              rows_v.at[b1_], ssem).wait()
        pltpu.sync_copy(rows_v.at[b1_], acc_sh.at[dst_v.at[j0 + 1]],
                        add=True)
        return carry

    lax.fori_loop(0, CPW // 2, step, 0)
    plsc.subcore_barrier()

    pltpu.sync_copy(acc_sh.at[pl.ds(s * ZERO_ROWS, ZERO_ROWS)], out_v)
    c = wid % NC
    pltpu.sync_copy(out_v, out_hbm.at[c, pl.ds(s * ZERO_ROWS, ZERO_ROWS)])


def _seg_sum1_body(p_hbm, src_hbm, dst_hbm, zeros_hbm, out_hbm,
                   src_v, dst_v, rows_v, out_v, acc_sh, tab_sh, gsem, ssem):
    c = lax.axis_index("c")
    s = lax.axis_index("s")
    pltpu.sync_copy(zeros_hbm, out_v)
    pltpu.sync_copy(out_v, acc_sh.at[pl.ds(s * ZERO_ROWS, ZERO_ROWS)])
    pltpu.sync_copy(p_hbm.at[pl.ds(s * P_ROWS, P_ROWS), pl.ds(0, HIDDEN)],
                    tab_sh.at[pl.ds(s * P_ROWS, P_ROWS)])
    _edge_phase(s, s * NC + c, src_hbm, dst_hbm, out_hbm,
                src_v, dst_v, rows_v, out_v, acc_sh, tab_sh, gsem, ssem)


def _seg_sum2_body(parts_hbm, b1_hbm, src_hbm, dst_hbm, zeros_hbm, out_hbm,
                   src_v, dst_v, rows_v, out_v, a_v, b_v, b1_v,
                   acc_sh, tab_sh, gsem, ssem):
    c = lax.axis_index("c")
    s = lax.axis_index("s")
    pltpu.sync_copy(zeros_hbm, out_v)
    pltpu.sync_copy(out_v, acc_sh.at[pl.ds(s * ZERO_ROWS, ZERO_ROWS)])

    pltpu.sync_copy(parts_hbm.at[0, pl.ds(s * P_ROWS, P_ROWS)], a_v)
    pltpu.sync_copy(parts_hbm.at[1, pl.ds(s * P_ROWS, P_ROWS)], b_v)
    pltpu.sync_copy(b1_hbm, b1_v)
    b1vec = b1_v[...]

    def rowf(r, carry):
        out_v[r, :] = jnp.maximum(a_v[r, :] + b_v[r, :] + b1vec, 0.0)
        return carry

    lax.fori_loop(0, P_ROWS, rowf, 0)
    pltpu.sync_copy(out_v.at[pl.ds(0, P_ROWS)],
                    tab_sh.at[pl.ds(s * P_ROWS, P_ROWS)])
    _edge_phase(s, s * NC + c, src_hbm, dst_hbm, out_hbm,
                src_v, dst_v, rows_v, out_v, acc_sh, tab_sh, gsem, ssem)


_COMMON_SCRATCH = [
    pltpu.VMEM((CPW, CH), jnp.int32),
    pltpu.VMEM((CPW, CH), jnp.int32),
    pltpu.VMEM((NBUF, CH, HIDDEN), jnp.float32),
    pltpu.VMEM((ZERO_ROWS, HIDDEN), jnp.float32),
]
_SHARED_SCRATCH = [
    pltpu.VMEM_SHARED((N_ACC, HIDDEN), jnp.float32),
    pltpu.VMEM_SHARED((N_NODES, HIDDEN), jnp.float32),
    pltpu.SemaphoreType.DMA,
    pltpu.SemaphoreType.DMA,
]

_seg_sum1 = pl.kernel(
    _seg_sum1_body,
    out_type=jax.ShapeDtypeStruct((NC, N_ACC, HIDDEN), jnp.float32),
    mesh=_SC_MESH,
    scratch_types=_COMMON_SCRATCH + _SHARED_SCRATCH,
    compiler_params=pltpu.CompilerParams(use_tc_tiling_on_sc=False),
)

_seg_sum2 = pl.kernel(
    _seg_sum2_body,
    out_type=jax.ShapeDtypeStruct((NC, N_ACC, HIDDEN), jnp.float32),
    mesh=_SC_MESH,
    scratch_types=_COMMON_SCRATCH + [
        pltpu.VMEM((P_ROWS, HIDDEN), jnp.float32),
        pltpu.VMEM((P_ROWS, HIDDEN), jnp.float32),
        pltpu.VMEM((HIDDEN,), jnp.float32),
    ] + _SHARED_SCRATCH,
    compiler_params=pltpu.CompilerParams(use_tc_tiling_on_sc=False),
)


def kernel(features, edge_index, W1, b1, W2, b2):
    src = edge_index[0].astype(jnp.int32)
    dst = edge_index[1].astype(jnp.int32)
    pad = E_PAD - N_EDGES
    src2d = jnp.concatenate(
        [src, jnp.zeros((pad,), jnp.int32)]).reshape(N_CHUNKS, CH)
    dst2d = jnp.concatenate(
        [dst, jnp.full((pad,), N_NODES, jnp.int32)]).reshape(N_CHUNKS, CH)
    zeros = jnp.zeros((ZERO_ROWS, HIDDEN), jnp.float32)
    W2p = jnp.pad(W2, ((0, 0), (0, HIDDEN - OUT_FEATS)))
    b2p = jnp.pad(b2, (0, HIDDEN - OUT_FEATS))

    W2blk = jnp.kron(jnp.eye(8, dtype=jnp.float32), W2p)
    b2t = jnp.tile(b2p, 8)

    W1p = jnp.pad(W1, ((0, 0), (0, 128 - HIDDEN)))
    p1 = _project1(features, W1p)
    parts1 = _seg_sum1(p1, src2d, dst2d, zeros)
    parts2 = _seg_sum2(parts1, b1, src2d, dst2d, zeros)
    parts_pk = parts2.reshape(NC, PK_ROWS, 128)
    out_pk = _final(parts_pk, W2blk, b2t)
    return out_pk.reshape(N_ACC, HIDDEN)[:N_NODES, :OUT_FEATS]
